# Initial kernel scaffold; baseline (speedup 1.0000x reference)
import jax
import jax.numpy as jnp
from jax.experimental import pallas as pl


def kernel(x, edge_index, batch, W1, b1, W2, b2, W3, b3, g1, be1, g2, be2, g3, be3, Wc1, bc1, Wc2, bc2, Wc3, bc3):
    raise NotImplementedError("write your pallas kernel here")


# speedup vs baseline: 2.3381x; 2.3381x over previous
import functools

import jax
import jax.numpy as jnp
from jax.experimental import pallas as pl
from jax.experimental.pallas import tpu as pltpu

_N = 10000
_E = 160000
_G = 16
_D = 256
_H = 256
_OUT = 10
_EPS = 1e-5

_R = 2000
_NB = _N // _R


def _mm_scale_body(dinv_ref, h_ref, w_ref, out_ref):
    out_ref[...] = dinv_ref[...] * jnp.dot(
        h_ref[...], w_ref[...], preferred_element_type=jnp.float32)


def _mm_scale(h, W, dinv):
    return pl.pallas_call(
        _mm_scale_body,
        grid=(_NB,),
        in_specs=[
            pl.BlockSpec((_R, 1), lambda i: (i, 0)),
            pl.BlockSpec((_R, _D), lambda i: (i, 0)),
            pl.BlockSpec((_D, _H), lambda i: (0, 0)),
        ],
        out_specs=pl.BlockSpec((_R, _H), lambda i: (i, 0)),
        out_shape=jax.ShapeDtypeStruct((_N, _H), jnp.float32),
    )(dinv, h, W)


def _stats_body(dinv_ref, acc_ref, conv_ref, st_ref, s1, s2):
    i = pl.program_id(0)
    conv = dinv_ref[...] * acc_ref[...]
    conv_ref[...] = conv

    @pl.when(i == 0)
    def _():
        s1[...] = jnp.zeros_like(s1)
        s2[...] = jnp.zeros_like(s2)

    s1[...] += jnp.sum(conv, axis=0, keepdims=True)
    s2[...] += jnp.sum(conv * conv, axis=0, keepdims=True)

    @pl.when(i == _NB - 1)
    def _():
        st_ref[0:1, :] = s1[...]
        st_ref[1:2, :] = s2[...]


def _stats(acc, dinv):
    return pl.pallas_call(
        _stats_body,
        grid=(_NB,),
        in_specs=[
            pl.BlockSpec((_R, 1), lambda i: (i, 0)),
            pl.BlockSpec((_R, _H), lambda i: (i, 0)),
        ],
        out_specs=[
            pl.BlockSpec((_R, _H), lambda i: (i, 0)),
            pl.BlockSpec((2, _H), lambda i: (0, 0)),
        ],
        out_shape=[
            jax.ShapeDtypeStruct((_N, _H), jnp.float32),
            jax.ShapeDtypeStruct((2, _H), jnp.float32),
        ],
        scratch_shapes=[
            pltpu.VMEM((1, _H), jnp.float32),
            pltpu.VMEM((1, _H), jnp.float32),
        ],
    )(dinv, acc)


def _bn_mm_body(st_ref, g_ref, be_ref, dinv_ref, conv_ref, w_ref, out_ref):
    m = st_ref[0:1, :] / _N
    var = st_ref[1:2, :] / _N - m * m
    a = jax.lax.rsqrt(var + _EPS) * g_ref[...]
    c = be_ref[...] - m * a
    hn = jnp.maximum(conv_ref[...] * a + c, 0.0)
    out_ref[...] = dinv_ref[...] * jnp.dot(
        hn, w_ref[...], preferred_element_type=jnp.float32)


def _bn_mm(st, g, be, dinv, conv, W):
    return pl.pallas_call(
        _bn_mm_body,
        grid=(_NB,),
        in_specs=[
            pl.BlockSpec((2, _H), lambda i: (0, 0)),
            pl.BlockSpec((1, _H), lambda i: (0, 0)),
            pl.BlockSpec((1, _H), lambda i: (0, 0)),
            pl.BlockSpec((_R, 1), lambda i: (i, 0)),
            pl.BlockSpec((_R, _H), lambda i: (i, 0)),
            pl.BlockSpec((_H, _H), lambda i: (0, 0)),
        ],
        out_specs=pl.BlockSpec((_R, _H), lambda i: (i, 0)),
        out_shape=jax.ShapeDtypeStruct((_N, _H), jnp.float32),
    )(st, g[None, :], be[None, :], dinv, conv, W)


def _pool_cls_body(st_ref, g_ref, be_ref, batch_ref, conv_ref,
                   wc1_ref, bc1_ref, wc2_ref, bc2_ref, wc3_ref, bc3_ref,
                   out_ref, s_acc, mx_acc, cnt_acc):
    i = pl.program_id(0)
    m = st_ref[0:1, :] / _N
    var = st_ref[1:2, :] / _N - m * m
    a = jax.lax.rsqrt(var + _EPS) * g_ref[...]
    c = be_ref[...] - m * a
    hn = jnp.maximum(conv_ref[...] * a + c, 0.0)
    b = batch_ref[...]
    gids = jax.lax.broadcasted_iota(jnp.int32, (1, _G), 1).astype(jnp.float32)
    onehot = (b == gids).astype(jnp.float32)

    @pl.when(i == 0)
    def _():
        s_acc[...] = jnp.zeros_like(s_acc)
        cnt_acc[...] = jnp.zeros_like(cnt_acc)
        mx_acc[...] = jnp.full_like(mx_acc, -jnp.inf)

    s_acc[...] += jax.lax.dot_general(
        onehot, hn, (((0,), (0,)), ((), ())),
        preferred_element_type=jnp.float32)
    cnt_acc[...] += jax.lax.dot_general(
        onehot, jnp.ones((_R, 1), jnp.float32), (((0,), (0,)), ((), ())),
        preferred_element_type=jnp.float32)
    for gi in range(_G):
        mg = jnp.max(jnp.where(b == float(gi), hn, -jnp.inf),
                     axis=0, keepdims=True)
        mx_acc[gi:gi + 1, :] = jnp.maximum(mx_acc[gi:gi + 1, :], mg)

    @pl.when(i == _NB - 1)
    def _():
        s = s_acc[...]
        mean = s / jnp.maximum(cnt_acc[...], 1.0)
        mx = mx_acc[...]
        z1 = jnp.maximum(
            jnp.dot(mean, wc1_ref[0:_H, :], preferred_element_type=jnp.float32)
            + jnp.dot(mx, wc1_ref[_H:2 * _H, :], preferred_element_type=jnp.float32)
            + jnp.dot(s, wc1_ref[2 * _H:3 * _H, :], preferred_element_type=jnp.float32)
            + bc1_ref[...], 0.0)
        z2 = jnp.maximum(
            jnp.dot(z1, wc2_ref[...], preferred_element_type=jnp.float32)
            + bc2_ref[...], 0.0)
        out_ref[...] = jnp.dot(
            z2, wc3_ref[...], preferred_element_type=jnp.float32) + bc3_ref[...]


def _pool_cls(conv, st, g, be, batch_f, Wc1, bc1, Wc2, bc2, Wc3, bc3):
    return pl.pallas_call(
        _pool_cls_body,
        grid=(_NB,),
        in_specs=[
            pl.BlockSpec((2, _H), lambda i: (0, 0)),
            pl.BlockSpec((1, _H), lambda i: (0, 0)),
            pl.BlockSpec((1, _H), lambda i: (0, 0)),
            pl.BlockSpec((_R, 1), lambda i: (i, 0)),
            pl.BlockSpec((_R, _H), lambda i: (i, 0)),
            pl.BlockSpec((3 * _H, 2 * _H), lambda i: (0, 0)),
            pl.BlockSpec((1, 2 * _H), lambda i: (0, 0)),
            pl.BlockSpec((2 * _H, _H), lambda i: (0, 0)),
            pl.BlockSpec((1, _H), lambda i: (0, 0)),
            pl.BlockSpec((_H, _OUT), lambda i: (0, 0)),
            pl.BlockSpec((1, _OUT), lambda i: (0, 0)),
        ],
        out_specs=pl.BlockSpec((_G, _OUT), lambda i: (0, 0)),
        out_shape=jax.ShapeDtypeStruct((_G, _OUT), jnp.float32),
        scratch_shapes=[
            pltpu.VMEM((_G, _H), jnp.float32),
            pltpu.VMEM((_G, _H), jnp.float32),
            pltpu.VMEM((_G, 1), jnp.float32),
        ],
    )(st, g[None, :], be[None, :], batch_f, conv,
      Wc1, bc1[None, :], Wc2, bc2[None, :], Wc3, bc3[None, :])


def kernel(x, edge_index, batch, W1, b1, W2, b2, W3, b3,
           g1, be1, g2, be2, g3, be3, Wc1, bc1, Wc2, bc2, Wc3, bc3):
    src = edge_index[0]
    dst = edge_index[1]
    deg = jnp.zeros((_N,), jnp.float32).at[dst].add(1.0) + 1.0
    dinv = jax.lax.rsqrt(deg)[:, None]
    batch_f = batch.astype(jnp.float32)[:, None]

    hs = _mm_scale(x, W1, dinv)
    for (W_next, g, be) in ((W2, g1, be1), (W3, g2, be2)):
        acc = hs.at[dst].add(hs[src])
        conv, st = _stats(acc, dinv)
        hs = _bn_mm(st, g, be, dinv, conv, W_next)
    acc = hs.at[dst].add(hs[src])
    conv, st = _stats(acc, dinv)
    return _pool_cls(conv, st, g3, be3, batch_f,
                     Wc1, bc1, Wc2, bc2, Wc3, bc3)


# speedup vs baseline: 7.0093x; 2.9978x over previous
import functools

import jax
import jax.numpy as jnp
from jax import lax
from jax.experimental import pallas as pl
from jax.experimental.pallas import tpu as pltpu
from jax.experimental.pallas import tpu_sc as plsc

_N = 10000
_E = 160000
_G = 16
_D = 256
_H = 256
_HH = 128
_OUT = 10
_EPS = 1e-5

_R = 2000
_NB = _N // _R

_NC = 2
_NS = 16

_ES = _E // _NS
_K = 80
_NCHUNK = _ES // _K
_RS = _N // _NS

_EC = _E // _NC
_ESD = _EC // _NS
_KD = 40
_NCHUNKD = _ESD // _KD
_RD = 624


def _sc_mesh():
    return plsc.VectorSubcoreMesh(core_axis_name="c", subcore_axis_name="s")


def _degree_body(dst_hbm, out_hbm, deg_sh, idx_v, ones_v, zb_v):
    c = lax.axis_index("c")
    s = lax.axis_index("s")

    for j in range(48 // 16):
        ones_v[pl.ds(j * 16, 16)] = jnp.ones((16,), jnp.float32)
    for j in range(640 // 16):
        zb_v[pl.ds(j * 16, 16)] = jnp.zeros((16,), jnp.float32)

    @pl.when(s < _NS - 1)
    def _():
        pltpu.sync_copy(zb_v.at[pl.ds(0, _RD)], deg_sh.at[pl.ds(s * _RD, _RD)])

    @pl.when(s == _NS - 1)
    def _():
        pltpu.sync_copy(zb_v, deg_sh.at[pl.ds((_NS - 1) * _RD, 640)])

    plsc.subcore_barrier()

    def body(g, carry):
        base = c * _EC + s * _ESD + g * _KD
        pltpu.sync_copy(dst_hbm.at[pl.ds(base, _KD)], idx_v)
        pltpu.sync_copy(ones_v.at[pl.ds(0, _KD)], deg_sh.at[idx_v], add=True)
        return carry

    lax.fori_loop(0, _NCHUNKD, body, 0)

    plsc.subcore_barrier()

    @pl.when(s < _NS - 1)
    def _():
        pltpu.sync_copy(deg_sh.at[pl.ds(s * _RD, _RD)], zb_v.at[pl.ds(0, _RD)])
        pltpu.sync_copy(zb_v.at[pl.ds(0, _RD)],
                        out_hbm.at[pl.ds(c * _N + s * _RD, _RD)])

    @pl.when(s == _NS - 1)
    def _():
        pltpu.sync_copy(deg_sh.at[pl.ds((_NS - 1) * _RD, 640)], zb_v)
        pltpu.sync_copy(zb_v,
                        out_hbm.at[pl.ds(c * _N + (_NS - 1) * _RD, 640)])


def _sc_degree(dst):
    k = pl.kernel(
        _degree_body,
        out_type=jax.ShapeDtypeStruct((_NC * _N,), jnp.float32),
        mesh=_sc_mesh(),
        scratch_types=[
            pltpu.VMEM_SHARED((_N,), jnp.float32),
            pltpu.VMEM((_KD,), jnp.int32),
            pltpu.VMEM((48,), jnp.float32),
            pltpu.VMEM((640,), jnp.float32),
        ],
    )
    return k(dst)


_RI = _RS // 5


def _scatter_body(hs_hbm, src_hbm, dst_hbm, out_hbm,
                  acc_sh, src_v, dst_v, rows_v, stage_v, sem):
    c = lax.axis_index("c")
    s = lax.axis_index("s")

    for k in range(5):
        r0 = s * _RS + k * _RI
        pltpu.sync_copy(hs_hbm.at[pl.ds(c * _N + r0, _RI)], stage_v)
        pltpu.sync_copy(stage_v, acc_sh.at[pl.ds(r0, _RI)])
    plsc.subcore_barrier()

    off = c * _N

    def body(g, carry):
        base = s * _ES + g * _K
        pltpu.sync_copy(src_hbm.at[pl.ds(base, _K)], src_v)
        pltpu.sync_copy(dst_hbm.at[pl.ds(base, _K)], dst_v)
        for j in range(_K // 16):
            src_v[pl.ds(j * 16, 16)] = src_v[pl.ds(j * 16, 16)] + off
        pltpu.async_copy(hs_hbm.at[src_v], rows_v, sem).wait()
        pltpu.sync_copy(rows_v, acc_sh.at[dst_v], add=True)
        return carry

    lax.fori_loop(0, _NCHUNK, body, 0)

    plsc.subcore_barrier()
    for k in range(5):
        r0 = s * _RS + k * _RI
        pltpu.sync_copy(acc_sh.at[pl.ds(r0, _RI)], stage_v)
        pltpu.sync_copy(stage_v, out_hbm.at[pl.ds(c * _N + r0, _RI)])


def _sc_scatter(hs, src, dst):
    k = pl.kernel(
        _scatter_body,
        out_type=jax.ShapeDtypeStruct((_NC * _N, _HH), jnp.float32),
        mesh=_sc_mesh(),
        scratch_types=[
            pltpu.VMEM_SHARED((_N, _HH), jnp.float32),
            pltpu.VMEM((_K,), jnp.int32),
            pltpu.VMEM((_K,), jnp.int32),
            pltpu.VMEM((_K, _HH), jnp.float32),
            pltpu.VMEM((_RS // 5, _HH), jnp.float32),
            pltpu.SemaphoreType.DMA,
        ],
        compiler_params=pltpu.CompilerParams(use_tc_tiling_on_sc=False),
    )
    return k(hs, src, dst)


def _mm1_body(deg0_ref, deg1_ref, h_ref, w_ref, hs_ref, dinv_ref):
    dinv = lax.rsqrt(deg0_ref[...] + deg1_ref[...] + 1.0)
    dinv_ref[...] = dinv
    mm = jnp.dot(h_ref[...], w_ref[...], preferred_element_type=jnp.float32)
    hs = dinv * mm
    hs_ref[0, :, :] = hs[:, 0:_HH]
    hs_ref[1, :, :] = hs[:, _HH:_H]


def _mm1(deg2, h, W):
    return pl.pallas_call(
        _mm1_body,
        grid=(_NB,),
        in_specs=[
            pl.BlockSpec((_R, 1), lambda i: (i, 0)),
            pl.BlockSpec((_R, 1), lambda i: (i + _NB, 0)),
            pl.BlockSpec((_R, _D), lambda i: (i, 0)),
            pl.BlockSpec((_D, _H), lambda i: (0, 0)),
        ],
        out_specs=[
            pl.BlockSpec((2, _R, _HH), lambda i: (0, i, 0)),
            pl.BlockSpec((_R, 1), lambda i: (i, 0)),
        ],
        out_shape=[
            jax.ShapeDtypeStruct((2, _N, _HH), jnp.float32),
            jax.ShapeDtypeStruct((_N, 1), jnp.float32),
        ],
    )(deg2, deg2, h, W)


def _stats_body(dinv_ref, acc_ref, conv_ref, st_ref, s1, s2):
    i = pl.program_id(0)
    conv = dinv_ref[...] * jnp.concatenate(
        [acc_ref[0, :, :], acc_ref[1, :, :]], axis=1)
    conv_ref[...] = conv

    @pl.when(i == 0)
    def _():
        s1[...] = jnp.zeros_like(s1)
        s2[...] = jnp.zeros_like(s2)

    s1[...] += jnp.sum(conv, axis=0, keepdims=True)
    s2[...] += jnp.sum(conv * conv, axis=0, keepdims=True)

    @pl.when(i == _NB - 1)
    def _():
        st_ref[0:1, :] = s1[...]
        st_ref[1:2, :] = s2[...]


def _stats(dinv, acc):
    return pl.pallas_call(
        _stats_body,
        grid=(_NB,),
        in_specs=[
            pl.BlockSpec((_R, 1), lambda i: (i, 0)),
            pl.BlockSpec((2, _R, _HH), lambda i: (0, i, 0)),
        ],
        out_specs=[
            pl.BlockSpec((_R, _H), lambda i: (i, 0)),
            pl.BlockSpec((2, _H), lambda i: (0, 0)),
        ],
        out_shape=[
            jax.ShapeDtypeStruct((_N, _H), jnp.float32),
            jax.ShapeDtypeStruct((2, _H), jnp.float32),
        ],
        scratch_shapes=[
            pltpu.VMEM((1, _H), jnp.float32),
            pltpu.VMEM((1, _H), jnp.float32),
        ],
    )(dinv, acc)


def _bn_mm_body(st_ref, g_ref, be_ref, dinv_ref, conv_ref, w_ref, hs_ref):
    m = st_ref[0:1, :] / _N
    var = st_ref[1:2, :] / _N - m * m
    a = lax.rsqrt(var + _EPS) * g_ref[...]
    c = be_ref[...] - m * a
    hn = jnp.maximum(conv_ref[...] * a + c, 0.0)
    hs = dinv_ref[...] * jnp.dot(hn, w_ref[...],
                                 preferred_element_type=jnp.float32)
    hs_ref[0, :, :] = hs[:, 0:_HH]
    hs_ref[1, :, :] = hs[:, _HH:_H]


def _bn_mm(st, g, be, dinv, conv, W):
    return pl.pallas_call(
        _bn_mm_body,
        grid=(_NB,),
        in_specs=[
            pl.BlockSpec((2, _H), lambda i: (0, 0)),
            pl.BlockSpec((1, _H), lambda i: (0, 0)),
            pl.BlockSpec((1, _H), lambda i: (0, 0)),
            pl.BlockSpec((_R, 1), lambda i: (i, 0)),
            pl.BlockSpec((_R, _H), lambda i: (i, 0)),
            pl.BlockSpec((_H, _H), lambda i: (0, 0)),
        ],
        out_specs=pl.BlockSpec((2, _R, _HH), lambda i: (0, i, 0)),
        out_shape=jax.ShapeDtypeStruct((2, _N, _HH), jnp.float32),
    )(st, g[None, :], be[None, :], dinv, conv, W)


def _pool_cls_body(st_ref, g_ref, be_ref, batch_ref, dinv_ref, acc_ref,
                   wc1_ref, bc1_ref, wc2_ref, bc2_ref, wc3_ref, bc3_ref,
                   out_ref, s_acc, mx_acc, cnt_acc):
    i = pl.program_id(0)
    m = st_ref[0:1, :] / _N
    var = st_ref[1:2, :] / _N - m * m
    a = lax.rsqrt(var + _EPS) * g_ref[...]
    c = be_ref[...] - m * a
    conv = dinv_ref[...] * jnp.concatenate(
        [acc_ref[0, :, :], acc_ref[1, :, :]], axis=1)
    hn = jnp.maximum(conv * a + c, 0.0)
    b = batch_ref[...]
    gids = jax.lax.broadcasted_iota(jnp.int32, (1, _G), 1).astype(jnp.float32)
    onehot = (b == gids).astype(jnp.float32)

    @pl.when(i == 0)
    def _():
        s_acc[...] = jnp.zeros_like(s_acc)
        cnt_acc[...] = jnp.zeros_like(cnt_acc)
        mx_acc[...] = jnp.full_like(mx_acc, -jnp.inf)

    s_acc[...] += jax.lax.dot_general(
        onehot, hn, (((0,), (0,)), ((), ())),
        preferred_element_type=jnp.float32)
    cnt_acc[...] += jax.lax.dot_general(
        onehot, jnp.ones((_R, 1), jnp.float32), (((0,), (0,)), ((), ())),
        preferred_element_type=jnp.float32)
    for gi in range(_G):
        mg = jnp.max(jnp.where(b == float(gi), hn, -jnp.inf),
                     axis=0, keepdims=True)
        mx_acc[gi:gi + 1, :] = jnp.maximum(mx_acc[gi:gi + 1, :], mg)

    @pl.when(i == _NB - 1)
    def _():
        s = s_acc[...]
        mean = s / jnp.maximum(cnt_acc[...], 1.0)
        mx = mx_acc[...]
        z1 = jnp.maximum(
            jnp.dot(mean, wc1_ref[0:_H, :], preferred_element_type=jnp.float32)
            + jnp.dot(mx, wc1_ref[_H:2 * _H, :], preferred_element_type=jnp.float32)
            + jnp.dot(s, wc1_ref[2 * _H:3 * _H, :], preferred_element_type=jnp.float32)
            + bc1_ref[...], 0.0)
        z2 = jnp.maximum(
            jnp.dot(z1, wc2_ref[...], preferred_element_type=jnp.float32)
            + bc2_ref[...], 0.0)
        out_ref[...] = jnp.dot(
            z2, wc3_ref[...], preferred_element_type=jnp.float32) + bc3_ref[...]


def _pool_cls(st, g, be, batch_f, dinv, acc, Wc1, bc1, Wc2, bc2, Wc3, bc3):
    return pl.pallas_call(
        _pool_cls_body,
        grid=(_NB,),
        in_specs=[
            pl.BlockSpec((2, _H), lambda i: (0, 0)),
            pl.BlockSpec((1, _H), lambda i: (0, 0)),
            pl.BlockSpec((1, _H), lambda i: (0, 0)),
            pl.BlockSpec((_R, 1), lambda i: (i, 0)),
            pl.BlockSpec((_R, 1), lambda i: (i, 0)),
            pl.BlockSpec((2, _R, _HH), lambda i: (0, i, 0)),
            pl.BlockSpec((3 * _H, 2 * _H), lambda i: (0, 0)),
            pl.BlockSpec((1, 2 * _H), lambda i: (0, 0)),
            pl.BlockSpec((2 * _H, _H), lambda i: (0, 0)),
            pl.BlockSpec((1, _H), lambda i: (0, 0)),
            pl.BlockSpec((_H, _OUT), lambda i: (0, 0)),
            pl.BlockSpec((1, _OUT), lambda i: (0, 0)),
        ],
        out_specs=pl.BlockSpec((_G, _OUT), lambda i: (0, 0)),
        out_shape=jax.ShapeDtypeStruct((_G, _OUT), jnp.float32),
        scratch_shapes=[
            pltpu.VMEM((_G, _H), jnp.float32),
            pltpu.VMEM((_G, _H), jnp.float32),
            pltpu.VMEM((_G, 1), jnp.float32),
        ],
    )(st, g[None, :], be[None, :], batch_f, dinv, acc,
      Wc1, bc1[None, :], Wc2, bc2[None, :], Wc3, bc3[None, :])


def kernel(x, edge_index, batch, W1, b1, W2, b2, W3, b3,
           g1, be1, g2, be2, g3, be3, Wc1, bc1, Wc2, bc2, Wc3, bc3):
    src = edge_index[0]
    dst = edge_index[1]
    batch_f = batch.astype(jnp.float32)[:, None]

    deg2 = _sc_degree(dst).reshape(_NC * _N, 1)
    hs3, dinv = _mm1(deg2, x, W1)

    for (W_next, g, be) in ((W2, g1, be1), (W3, g2, be2)):
        acc = _sc_scatter(hs3.reshape(_NC * _N, _HH), src, dst)
        conv, st = _stats(dinv, acc.reshape(_NC, _N, _HH))
        hs3 = _bn_mm(st, g, be, dinv, conv, W_next)

    acc = _sc_scatter(hs3.reshape(_NC * _N, _HH), src, dst)
    _, st = _stats(dinv, acc.reshape(_NC, _N, _HH))
    return _pool_cls(st, g3, be3, batch_f, dinv, acc.reshape(_NC, _N, _HH),
                     Wc1, bc1, Wc2, bc2, Wc3, bc3)
